# baseline (speedup 1.0000x reference)
import functools
import math

import jax
import jax.numpy as jnp
from jax import lax
from jax.experimental import pallas as pl
from jax.experimental.pallas import tpu as pltpu
from jax.experimental.pallas import tpu_sc as plsc

D_MODEL = 64
SCALE = math.sqrt(D_MODEL)

_NUM_CORES = 2
_NUM_SUBCORES = 16
_NUM_WORKERS = _NUM_CORES * _NUM_SUBCORES
_LANES = 16

_CHUNK = 128
_NBUF = 4


def _make_lookup(n_rows: int, n_batch: int):
    assert n_batch == _NUM_WORKERS * _CHUNK
    nchunk = n_rows
    assert nchunk % _NBUF == 0

    mesh = plsc.VectorSubcoreMesh(core_axis_name="c", subcore_axis_name="s")

    @functools.partial(
        pl.kernel,
        mesh=mesh,
        compiler_params=pltpu.CompilerParams(
            use_tc_tiling_on_sc=False, needs_layout_passes=False),
        out_type=jax.ShapeDtypeStruct((n_rows, D_MODEL, n_batch), jnp.float32),
        scratch_types=[
            pltpu.VMEM((nchunk, _CHUNK), jnp.int32),
            pltpu.VMEM((_NBUF, _CHUNK, D_MODEL), jnp.float32),
            pltpu.VMEM((_NBUF, D_MODEL, _CHUNK), jnp.float32),
            [pltpu.SemaphoreType.DMA] * _NBUF,
            [pltpu.SemaphoreType.DMA] * _NBUF,
        ],
    )
    def lookup(idx_hbm, table_hbm, out_hbm, idx_v, rows_g, rows_s, gsems, ssems):
        wid = lax.axis_index("s") * _NUM_CORES + lax.axis_index("c")
        i0 = wid * _CHUNK

        pltpu.sync_copy(idx_hbm.at[:, pl.ds(i0, _CHUNK)], idx_v)

        def fire_gather(c, b):
            pltpu.async_copy(table_hbm.at[idx_v.at[c]], rows_g.at[b], gsems[b])

        for c in range(_NBUF - 1):
            fire_gather(c, c)

        lane = lax.iota(jnp.int32, _LANES)

        @pl.loop(0, nchunk, step=_NBUF)
        def _(go):
            for b in range(_NBUF):
                c = go + b
                bb = (b + _NBUF - 1) % _NBUF

                @pl.when(c + _NBUF - 1 < nchunk)
                def _():
                    fire_gather(c + _NBUF - 1, bb)

                pltpu.make_async_copy(
                    table_hbm.at[idx_v.at[c]], rows_g.at[b], gsems[b]).wait()

                @pl.when(go >= _NBUF)
                def _():
                    pltpu.make_async_copy(
                        rows_s.at[b],
                        out_hbm.at[c - _NBUF, :, pl.ds(i0, _CHUNK)],
                        ssems[b]).wait()

                @plsc.parallel_loop(0, D_MODEL, unroll=2)
                def _(d):
                    col = jnp.full((_LANES,), d, jnp.int32)
                    for q in range(_CHUNK // _LANES):
                        v = plsc.load_gather(
                            rows_g.at[b], [lane + (q * _LANES), col])
                        rows_s[b, d, pl.ds(q * _LANES, _LANES)] = v * SCALE

                pltpu.async_copy(
                    rows_s.at[b],
                    out_hbm.at[c, :, pl.ds(i0, _CHUNK)],
                    ssems[b])

        for b in range(_NBUF):
            c = nchunk - _NBUF + b
            pltpu.make_async_copy(
                rows_s.at[b],
                out_hbm.at[c, :, pl.ds(i0, _CHUNK)],
                ssems[b]).wait()

    return lookup


def kernel(x, weight):
    n_i, n_j = x.shape
    xt = jnp.transpose(x).astype(jnp.int32)
    out_t = _make_lookup(n_j, n_i)(xt, weight)
    return jnp.transpose(out_t, (2, 0, 1))


# speedup vs baseline: 1.0443x; 1.0443x over previous
import functools
import math

import jax
import jax.numpy as jnp
from jax import lax
from jax.experimental import pallas as pl
from jax.experimental.pallas import tpu as pltpu
from jax.experimental.pallas import tpu_sc as plsc

D_MODEL = 64
SCALE = math.sqrt(D_MODEL)

_NUM_CORES = 2
_NUM_SUBCORES = 16
_NUM_WORKERS = _NUM_CORES * _NUM_SUBCORES
_LANES = 16

_CHUNK = 128
_NBUF = 4


def _make_lookup(n_rows: int, n_batch: int):
    assert n_batch == _NUM_WORKERS * _CHUNK
    nchunk = n_rows
    assert nchunk % _NBUF == 0

    mesh = plsc.VectorSubcoreMesh(core_axis_name="c", subcore_axis_name="s")

    @functools.partial(
        pl.kernel,
        mesh=mesh,
        compiler_params=pltpu.CompilerParams(
            use_tc_tiling_on_sc=False, needs_layout_passes=False),
        out_type=jax.ShapeDtypeStruct(
            (n_rows, _NUM_WORKERS, D_MODEL, _CHUNK), jnp.float32),
        scratch_types=[
            pltpu.VMEM((nchunk, _CHUNK), jnp.int32),
            pltpu.VMEM((_NBUF, _CHUNK, D_MODEL), jnp.float32),
            pltpu.VMEM((_NBUF, D_MODEL, _CHUNK), jnp.float32),
            [pltpu.SemaphoreType.DMA] * _NBUF,
            [pltpu.SemaphoreType.DMA] * _NBUF,
        ],
    )
    def lookup(idx_hbm, table_hbm, out_hbm, idx_v, rows_g, rows_s, gsems, ssems):
        wid = lax.axis_index("s") * _NUM_CORES + lax.axis_index("c")
        i0 = wid * _CHUNK

        pltpu.sync_copy(idx_hbm.at[:, pl.ds(i0, _CHUNK)], idx_v)

        def fire_gather(c, b):
            pltpu.async_copy(table_hbm.at[idx_v.at[c]], rows_g.at[b], gsems[b])

        for c in range(_NBUF - 1):
            fire_gather(c, c)

        lane = lax.iota(jnp.int32, _LANES)

        @pl.loop(0, nchunk, step=_NBUF)
        def _(go):
            for b in range(_NBUF):
                c = go + b
                bb = (b + _NBUF - 1) % _NBUF

                @pl.when(c + _NBUF - 1 < nchunk)
                def _():
                    fire_gather(c + _NBUF - 1, bb)

                pltpu.make_async_copy(
                    table_hbm.at[idx_v.at[c]], rows_g.at[b], gsems[b]).wait()

                @pl.when(go >= _NBUF)
                def _():
                    pltpu.make_async_copy(
                        rows_s.at[b],
                        out_hbm.at[c - _NBUF, wid],
                        ssems[b]).wait()

                @plsc.parallel_loop(0, D_MODEL, unroll=2)
                def _(d):
                    col = jnp.full((_LANES,), d, jnp.int32)
                    for q in range(_CHUNK // _LANES):
                        v = plsc.load_gather(
                            rows_g.at[b], [lane + (q * _LANES), col])
                        rows_s[b, d, pl.ds(q * _LANES, _LANES)] = v * SCALE

                pltpu.async_copy(
                    rows_s.at[b],
                    out_hbm.at[c, wid],
                    ssems[b])

        for b in range(_NBUF):
            c = nchunk - _NBUF + b
            pltpu.make_async_copy(
                rows_s.at[b],
                out_hbm.at[c, wid],
                ssems[b]).wait()

    return lookup


def kernel(x, weight):
    n_i, n_j = x.shape
    xt = jnp.transpose(x).astype(jnp.int32)
    out_p = _make_lookup(n_j, n_i)(xt, weight)
    return out_p.transpose(1, 3, 0, 2).reshape(n_i, n_j, D_MODEL)


# speedup vs baseline: 1.1177x; 1.0703x over previous
import functools
import math

import jax
import jax.numpy as jnp
from jax import lax
from jax.experimental import pallas as pl
from jax.experimental.pallas import tpu as pltpu
from jax.experimental.pallas import tpu_sc as plsc

D_MODEL = 64
SCALE = math.sqrt(D_MODEL)

_NUM_CORES = 2
_NUM_SUBCORES = 16
_NUM_WORKERS = _NUM_CORES * _NUM_SUBCORES
_LANES = 16

_CHUNK = 128
_PAIR = 2 * D_MODEL
_NBUF = 4


def _make_lookup(batch: int):
    assert batch % (_NUM_WORKERS * _CHUNK) == 0
    bpw = batch // _NUM_WORKERS
    nchunk = bpw // _CHUNK
    assert nchunk % _NBUF == 0

    mesh = plsc.VectorSubcoreMesh(core_axis_name="c", subcore_axis_name="s")

    @functools.partial(
        pl.kernel,
        mesh=mesh,
        compiler_params=pltpu.CompilerParams(use_tc_tiling_on_sc=True),
        out_type=jax.ShapeDtypeStruct((batch // 2, _PAIR), jnp.float32),
        scratch_types=[
            pltpu.VMEM((nchunk, _CHUNK), jnp.int32),
            pltpu.VMEM((_NBUF, _CHUNK), jnp.int32),
            pltpu.VMEM((_NBUF, _CHUNK, _PAIR), jnp.float32),
            pltpu.VMEM((_NBUF, D_MODEL, _PAIR), jnp.float32),
            pltpu.SMEM((_NBUF, _CHUNK), jnp.int32),
            [pltpu.SemaphoreType.DMA] * _NBUF,
            [pltpu.SemaphoreType.DMA] * _NBUF,
        ],
    )
    def lookup(idx_hbm, table_hbm, out_hbm, idx_v, idx2_v, rows_g, rows_s,
               hoff_s, gsems, ssems):
        wid = lax.axis_index("s") * _NUM_CORES + lax.axis_index("c")
        base = wid * bpw
        orow0 = pl.multiple_of(base // 2, 64)

        pltpu.sync_copy(
            idx_hbm.at[pl.ds(pl.multiple_of(wid * nchunk, 8), nchunk)], idx_v)

        def fire_gather(c, b):
            for q in range(_CHUNK // _LANES):
                sl = pl.ds(q * _LANES, _LANES)
                iv = idx_v[c, sl]
                idx2_v[b, sl] = lax.shift_right_logical(iv, 1)
                hv = (iv & 1) * D_MODEL
                for l in range(_LANES):
                    hoff_s[b, q * _LANES + l] = hv[l]
            pltpu.async_copy(table_hbm.at[idx2_v.at[b]], rows_g.at[b],
                             gsems[b])

        for c in range(_NBUF - 1):
            fire_gather(c, c)

        @pl.loop(0, nchunk, step=_NBUF)
        def _(go):
            for b in range(_NBUF):
                c = go + b
                bb = (b + _NBUF - 1) % _NBUF

                @pl.when(c + _NBUF - 1 < nchunk)
                def _():
                    fire_gather(c + _NBUF - 1, bb)

                pltpu.make_async_copy(
                    table_hbm.at[idx2_v.at[b]], rows_g.at[b], gsems[b]).wait()

                @pl.when(go >= _NBUF)
                def _():
                    pltpu.make_async_copy(
                        rows_s.at[b],
                        out_hbm.at[pl.ds(orow0 + (c - _NBUF) * (_CHUNK // 2),
                                         _CHUNK // 2)],
                        ssems[b]).wait()

                @plsc.parallel_loop(0, _CHUNK // 2, unroll=2)
                def _(k):
                    h0 = hoff_s[b, 2 * k]
                    h1 = hoff_s[b, 2 * k + 1]
                    for q in range(D_MODEL // _LANES):
                        rows_s[b, k, pl.ds(q * _LANES, _LANES)] = (
                            rows_g[b, 2 * k, pl.ds(h0 + q * _LANES, _LANES)]
                            * SCALE)
                        rows_s[b, k, pl.ds(D_MODEL + q * _LANES, _LANES)] = (
                            rows_g[b, 2 * k + 1,
                                   pl.ds(h1 + q * _LANES, _LANES)]
                            * SCALE)

                pltpu.async_copy(
                    rows_s.at[b],
                    out_hbm.at[pl.ds(orow0 + c * (_CHUNK // 2), _CHUNK // 2)],
                    ssems[b])

        for b in range(_NBUF):
            c = nchunk - _NBUF + b
            pltpu.make_async_copy(
                rows_s.at[b],
                out_hbm.at[pl.ds(orow0 + c * (_CHUNK // 2), _CHUNK // 2)],
                ssems[b]).wait()

    return lookup


def kernel(x, weight):
    batch = x.size
    xf = x.reshape(batch // _CHUNK, _CHUNK).astype(jnp.int32)
    wt = weight.reshape(weight.shape[0] // 2, _PAIR)
    out = _make_lookup(batch)(xf, wt)
    return out.reshape(*x.shape, D_MODEL)


# speedup vs baseline: 1.6314x; 1.4596x over previous
import functools
import math

import jax
import jax.numpy as jnp
from jax import lax
from jax.experimental import pallas as pl
from jax.experimental.pallas import tpu as pltpu
from jax.experimental.pallas import tpu_sc as plsc

D_MODEL = 64
SCALE = math.sqrt(D_MODEL)

_NUM_CORES = 2
_NUM_SUBCORES = 16
_NUM_WORKERS = _NUM_CORES * _NUM_SUBCORES
_LANES = 16

_CHUNK = 128
_PAIR = 2 * D_MODEL
_NGBUF = 4
_NSBUF = 2


def _make_lookup(n_rows: int, n_batch: int):
    assert n_batch == _NUM_WORKERS * _CHUNK
    nchunk = n_rows
    assert nchunk % _NGBUF == 0

    mesh = plsc.VectorSubcoreMesh(core_axis_name="c", subcore_axis_name="s")

    @functools.partial(
        pl.kernel,
        mesh=mesh,
        compiler_params=pltpu.CompilerParams(
            use_tc_tiling_on_sc=True, needs_layout_passes=False),
        out_type=jax.ShapeDtypeStruct((n_rows, D_MODEL, n_batch), jnp.float32),
        scratch_types=[
            pltpu.VMEM((nchunk, _CHUNK), jnp.int32),
            pltpu.VMEM((_NGBUF, _CHUNK), jnp.int32),
            pltpu.VMEM((_NGBUF, _CHUNK, _PAIR), jnp.float32),
            pltpu.VMEM((_CHUNK, D_MODEL), jnp.float32),
            pltpu.VMEM((_NSBUF, D_MODEL, _CHUNK), jnp.float32),
            pltpu.SMEM((_NGBUF, _CHUNK), jnp.int32),
            [pltpu.SemaphoreType.DMA] * _NGBUF,
            [pltpu.SemaphoreType.DMA] * _NSBUF,
        ],
    )
    def lookup(idx_hbm, table_hbm, out_hbm, idx_v, idx2_v, rows_g, skew_v,
               rows_s, hoff_s, gsems, ssems):
        wid = lax.axis_index("s") * _NUM_CORES + lax.axis_index("c")
        i0 = pl.multiple_of(wid * _CHUNK, 128)

        pltpu.sync_copy(idx_hbm.at[:, pl.ds(i0, _CHUNK)], idx_v)

        lane = lax.iota(jnp.int32, _LANES)

        def fire_gather(c, b):
            for q in range(_CHUNK // _LANES):
                sl = pl.ds(q * _LANES, _LANES)
                iv = idx_v[c, sl]
                idx2_v[b, sl] = lax.shift_right_logical(iv, 1)
                hv = (iv & 1) * D_MODEL
                for l in range(_LANES):
                    hoff_s[b, q * _LANES + l] = hv[l]
            pltpu.async_copy(table_hbm.at[idx2_v.at[b]], rows_g.at[b],
                             gsems[b])

        for c in range(_NGBUF - 1):
            fire_gather(c, c)

        @pl.loop(0, nchunk, step=_NGBUF)
        def _(go):
            for b in range(_NGBUF):
                c = go + b
                bs = b % _NSBUF
                bb = (b + _NGBUF - 1) % _NGBUF

                @pl.when(c + _NGBUF - 1 < nchunk)
                def _():
                    fire_gather(c + _NGBUF - 1, bb)

                pltpu.make_async_copy(
                    table_hbm.at[idx2_v.at[b]], rows_g.at[b], gsems[b]).wait()

                @pl.when(c >= _NSBUF)
                def _():
                    pltpu.make_async_copy(
                        rows_s.at[bs],
                        out_hbm.at[c - _NSBUF, :, pl.ds(i0, _CHUNK)],
                        ssems[bs]).wait()

                @plsc.parallel_loop(0, _CHUNK, unroll=2)
                def _(i):
                    h = hoff_s[b, i]
                    ib = jnp.full((_LANES,), i, jnp.int32)
                    for q in range(D_MODEL // _LANES):
                        col = (lane + (q * _LANES) + ib) & (D_MODEL - 1)
                        v = rows_g[b, i, pl.ds(h + q * _LANES, _LANES)]
                        plsc.store_scatter(skew_v, [ib, col], v * SCALE)

                @plsc.parallel_loop(0, D_MODEL, unroll=2)
                def _(d):
                    db = jnp.full((_LANES,), d, jnp.int32)
                    for q in range(_CHUNK // _LANES):
                        row = lane + (q * _LANES)
                        col = (db + row) & (D_MODEL - 1)
                        v = plsc.load_gather(skew_v, [row, col])
                        rows_s[bs, d, pl.ds(q * _LANES, _LANES)] = v

                pltpu.async_copy(
                    rows_s.at[bs],
                    out_hbm.at[c, :, pl.ds(i0, _CHUNK)],
                    ssems[bs])

        for bs in range(_NSBUF):
            c = nchunk - _NSBUF + bs
            pltpu.make_async_copy(
                rows_s.at[bs],
                out_hbm.at[c, :, pl.ds(i0, _CHUNK)],
                ssems[bs]).wait()

    return lookup


def kernel(x, weight):
    n_i, n_j = x.shape
    xt = jnp.transpose(x).astype(jnp.int32)
    wt = weight.reshape(weight.shape[0] // 2, _PAIR)
    out_t = _make_lookup(n_j, n_i)(xt, wt)
    return jnp.transpose(out_t, (2, 0, 1))


# speedup vs baseline: 1.6319x; 1.0003x over previous
import functools
import math

import jax
import jax.numpy as jnp
from jax import lax
from jax.experimental import pallas as pl
from jax.experimental.pallas import tpu as pltpu
from jax.experimental.pallas import tpu_sc as plsc

D_MODEL = 64
SCALE = math.sqrt(D_MODEL)

_NUM_CORES = 2
_NUM_SUBCORES = 16
_NUM_WORKERS = _NUM_CORES * _NUM_SUBCORES
_LANES = 16

_CHUNK = 128
_PAIR = 2 * D_MODEL
_NGBUF = 4
_NSBUF = 2


def _make_lookup(n_rows: int, n_batch: int):
    assert n_batch == _NUM_WORKERS * _CHUNK
    nchunk = n_rows
    assert nchunk % _NGBUF == 0

    mesh = plsc.VectorSubcoreMesh(core_axis_name="c", subcore_axis_name="s")

    @functools.partial(
        pl.kernel,
        mesh=mesh,
        compiler_params=pltpu.CompilerParams(
            use_tc_tiling_on_sc=True, needs_layout_passes=False),
        out_type=jax.ShapeDtypeStruct((n_rows, D_MODEL, n_batch), jnp.float32),
        scratch_types=[
            pltpu.VMEM((nchunk, _CHUNK), jnp.int32),
            pltpu.VMEM((_NGBUF, _CHUNK), jnp.int32),
            pltpu.VMEM((_NGBUF, _CHUNK, _PAIR), jnp.float32),
            pltpu.VMEM((_CHUNK, D_MODEL), jnp.float32),
            pltpu.VMEM((_NSBUF, D_MODEL, _CHUNK), jnp.float32),
            pltpu.SMEM((_NGBUF, _CHUNK), jnp.int32),
            [pltpu.SemaphoreType.DMA] * _NGBUF,
            [pltpu.SemaphoreType.DMA] * _NSBUF,
        ],
    )
    def lookup(idx_hbm, table_hbm, out_hbm, idx_v, idx2_v, rows_g, skew_v,
               rows_s, hoff_s, gsems, ssems):
        wid = lax.axis_index("s") * _NUM_CORES + lax.axis_index("c")
        i0 = pl.multiple_of(wid * _CHUNK, 128)

        pltpu.sync_copy(idx_hbm.at[:, pl.ds(i0, _CHUNK)], idx_v)

        lane = lax.iota(jnp.int32, _LANES)
        qlane = [lane + (q * _LANES) for q in range(_CHUNK // _LANES)]

        def fire_gather(c, b):
            for q in range(_CHUNK // _LANES):
                sl = pl.ds(q * _LANES, _LANES)
                iv = idx_v[c, sl]
                idx2_v[b, sl] = lax.shift_right_logical(iv, 1)
                hv = (iv & 1) * D_MODEL
                for l in range(_LANES):
                    hoff_s[b, q * _LANES + l] = hv[l]
            pltpu.async_copy(table_hbm.at[idx2_v.at[b]], rows_g.at[b],
                             gsems[b])

        for c in range(_NGBUF - 1):
            fire_gather(c, c)

        @pl.loop(0, nchunk, step=_NGBUF)
        def _(go):
            for b in range(_NGBUF):
                c = go + b
                bs = b % _NSBUF
                bb = (b + _NGBUF - 1) % _NGBUF

                @pl.when(c + _NGBUF - 1 < nchunk)
                def _():
                    fire_gather(c + _NGBUF - 1, bb)

                pltpu.make_async_copy(
                    table_hbm.at[idx2_v.at[b]], rows_g.at[b], gsems[b]).wait()

                @pl.when(c >= _NSBUF)
                def _():
                    pltpu.make_async_copy(
                        rows_s.at[bs],
                        out_hbm.at[c - _NSBUF, :, pl.ds(i0, _CHUNK)],
                        ssems[bs]).wait()

                @plsc.parallel_loop(0, _CHUNK, unroll=4)
                def _(i):
                    h = hoff_s[b, i]
                    ib = jnp.full((_LANES,), i, jnp.int32)
                    for q in range(D_MODEL // _LANES):
                        col = (qlane[q] + ib) & (D_MODEL - 1)
                        v = rows_g[b, i, pl.ds(h + q * _LANES, _LANES)]
                        plsc.store_scatter(skew_v, [ib, col], v * SCALE)

                @plsc.parallel_loop(0, D_MODEL, unroll=4)
                def _(d):
                    db = jnp.full((_LANES,), d, jnp.int32)
                    for q in range(_CHUNK // _LANES):
                        col = (db + qlane[q]) & (D_MODEL - 1)
                        v = plsc.load_gather(skew_v, [qlane[q], col])
                        rows_s[bs, d, pl.ds(q * _LANES, _LANES)] = v

                pltpu.async_copy(
                    rows_s.at[bs],
                    out_hbm.at[c, :, pl.ds(i0, _CHUNK)],
                    ssems[bs])

        for bs in range(_NSBUF):
            c = nchunk - _NSBUF + bs
            pltpu.make_async_copy(
                rows_s.at[bs],
                out_hbm.at[c, :, pl.ds(i0, _CHUNK)],
                ssems[bs]).wait()

    return lookup


def kernel(x, weight):
    n_i, n_j = x.shape
    xt = jnp.transpose(x).astype(jnp.int32)
    wt = weight.reshape(weight.shape[0] // 2, _PAIR)
    out_t = _make_lookup(n_j, n_i)(xt, wt)
    return jnp.transpose(out_t, (2, 0, 1))
